# baseline (speedup 1.0000x reference)
import functools

import jax
import jax.numpy as jnp
from jax import lax
from jax.experimental import pallas as pl
from jax.experimental.pallas import tpu as pltpu
from jax.experimental.pallas import tpu_sc as plsc

N_BINS = 15
_C = 100
_CP = 112
_NREAL = 50000
_NPAD = 51200
_BN_TC = 1600

_NW = 32
_NS = 16
_RPW = _NPAD // _NW
_CHR = 160
_NCH = _RPW // _CHR
_CW = _CHR * _CP

_SLOT = 16
_HCLS = _CP * _SLOT
_CFS_BASE = _HCLS
_CORR_BASE = 2 * _HCLS
_HTOT = 2 * _HCLS + 16 * _HCLS
_PUB = 3 * _HCLS
_MSL = _PUB // _NS


def _softmax_kernel(logits_ref, conf_ref):
    i = pl.program_id(0)
    x = logits_ref[...]
    bn, c = x.shape
    m = jnp.max(x, axis=1, keepdims=True)
    e = jnp.exp(x - m)
    s = jnp.sum(e, axis=1, keepdims=True)
    conf = e * (1.0 / s)
    row = jax.lax.broadcasted_iota(jnp.int32, (bn, c), 0) + i * bn
    conf = jnp.where(row < _NREAL, conf, 0.0)
    conf_ref[...] = jax.lax.pad(
        conf, jnp.float32(0.0), ((0, 0, 0), (0, _CP - c, 0)))


def _tc_conf(logits):
    n, c = logits.shape
    grid = _NPAD // _BN_TC
    return pl.pallas_call(
        _softmax_kernel,
        grid=(grid,),
        in_specs=[pl.BlockSpec((_BN_TC, c), lambda i: (i, 0))],
        out_specs=pl.BlockSpec((_BN_TC, _CP), lambda i: (i, 0)),
        out_shape=jax.ShapeDtypeStruct((_NPAD, _CP), jnp.float32),
        compiler_params=pltpu.CompilerParams(
            dimension_semantics=("arbitrary",)),
    )(logits)


def _bin_idx(v15):
    ti = v15.astype(jnp.int32)
    tf = ti.astype(jnp.float32)
    one = jnp.ones((16,), jnp.int32)
    return ti - jnp.where(tf == v15, one, jnp.zeros((16,), jnp.int32))


def _sc_kernel(conf_hbm, lbl_hbm, out_hbm,
               cb0, cb1, lb0, lb1, hist, tmp, macc, pub,
               sem0, sem1, shared):
    core = lax.axis_index("c")
    sub = lax.axis_index("s")
    wid = sub * 2 + core
    base_row = wid * _RPW

    iota = lax.iota(jnp.int32, 16)
    ones_f = jnp.ones((16,), jnp.float32)
    zeros_f = jnp.zeros((16,), jnp.float32)

    def _m16(i):
        return pl.multiple_of(i * 16, 16)

    def _z(i, _):
        hist[pl.ds(_m16(i), 16)] = zeros_f
        return 0
    lax.fori_loop(0, _HTOT // 16, _z, 0)

    cbufs = (cb0, cb1)
    lbufs = (lb0, lb1)
    sems = (sem0, sem1)

    def _start(k):
        h = pltpu.async_copy(
            conf_hbm.at[pl.ds(
                pl.multiple_of((base_row + k * _CHR) * _CP, 8), _CW)],
            cbufs[k % 2], sems[k % 2])
        pltpu.sync_copy(
            lbl_hbm.at[pl.ds(pl.multiple_of(base_row + k * _CHR, 8), _CHR)],
            lbufs[k % 2])
        return h

    addr_base = tuple(iota * 16 + (p * 256 + 1) for p in range(7))
    lane_off = iota * _HCLS + (_CORR_BASE + 1)

    handles = {0: _start(0)}
    for k in range(_NCH):
        handles.pop(k).wait()
        if k + 1 < _NCH:
            handles[k + 1] = _start(k + 1)
        cb = cbufs[k % 2]
        lb = lbufs[k % 2]

        def _row(j, _):
            base = pl.multiple_of(j * _CP, 8)
            for p in range(7):
                v = cb[pl.ds(base + p * 16, 16)]
                bi = _bin_idx(v * jnp.float32(N_BINS))
                addr = addr_base[p] + bi
                plsc.addupdate_scatter(hist, [addr], ones_f)
                plsc.addupdate_scatter(hist, [addr + _CFS_BASE], v)
            return 0
        lax.fori_loop(0, _CHR, _row, 0)

        def _grp(g, _):
            lvec = lb[pl.ds(pl.multiple_of(g * 16, 16), 16)]
            rloc = iota + jnp.full((16,), g * 16, jnp.int32)
            word = rloc * _CP + lvec
            cl = plsc.load_gather(cb, [word])
            bi = _bin_idx(cl * jnp.float32(N_BINS))
            caddr = lane_off + lvec * _SLOT + bi
            plsc.addupdate_scatter(hist, [caddr], ones_f)
            return 0
        lax.fori_loop(0, _CHR // 16, _grp, 0)

    def _cmp(i, _):
        o = _m16(i)
        pub[pl.ds(o, 16)] = hist[pl.ds(o, 16)]
        pub[pl.ds(_CFS_BASE + o, 16)] = hist[pl.ds(_CFS_BASE + o, 16)]
        acc = zeros_f
        for lane in range(16):
            acc = acc + hist[pl.ds(_CORR_BASE + lane * _HCLS + o, 16)]
        pub[pl.ds(2 * _HCLS + o, 16)] = acc
        return 0
    lax.fori_loop(0, _HCLS // 16, _cmp, 0)

    pltpu.sync_copy(pub, shared.at[pl.ds(pl.multiple_of(sub * _PUB, 8), _PUB)])
    plsc.subcore_barrier()

    sl = pl.multiple_of(sub * _MSL, 8)

    def _zm(i, _):
        macc[pl.ds(_m16(i), 16)] = zeros_f
        return 0
    lax.fori_loop(0, _MSL // 16, _zm, 0)

    def _mrg(w, _):
        off = pl.multiple_of(w * _PUB + sl, 8)
        pltpu.sync_copy(shared.at[pl.ds(off, _MSL)], tmp)

        def _add(i, _):
            macc[pl.ds(_m16(i), 16)] += tmp[pl.ds(_m16(i), 16)]
            return 0
        lax.fori_loop(0, _MSL // 16, _add, 0)
        return 0
    lax.fori_loop(0, _NS, _mrg, 0)
    plsc.subcore_barrier()

    pltpu.sync_copy(
        macc, out_hbm.at[pl.ds(pl.multiple_of(core * _PUB + sl, 8), _MSL)])


def _finish_kernel(hist_ref, labels_ref, out_ref, *, n_total):
    h = hist_ref[...]
    hs = h[0:1, :] + h[1:2, :]
    cnt = hs[:, 0:_HCLS]
    cfs = hs[:, _HCLS:2 * _HCLS]
    corr = hs[:, 2 * _HCLS:3 * _HCLS]
    lbl = labels_ref[...]
    maxl = jnp.max(lbl)
    num_classes = (maxl + 1).astype(jnp.float32)
    lane = jax.lax.broadcasted_iota(jnp.int32, (1, _HCLS), 1)
    slot = jax.lax.rem(lane, _SLOT)
    cls = jax.lax.div(lane, _SLOT)
    safe = jnp.maximum(cnt, 1.0)
    term = jnp.abs(cfs / safe - corr / safe) * cnt * jnp.float32(1.0 / n_total)
    ok = ((slot >= 1) & (cls < maxl + 1) & (cnt > 0.0)).astype(jnp.float32)
    out_ref[...] = jnp.sum(term * ok, keepdims=True) / num_classes


def _tc_finish(hist2, labels):
    n = labels.shape[0]
    out = pl.pallas_call(
        functools.partial(_finish_kernel, n_total=n),
        in_specs=[
            pl.BlockSpec((2, _PUB), lambda: (0, 0)),
            pl.BlockSpec((n, 1), lambda: (0, 0)),
        ],
        out_specs=pl.BlockSpec((1, 1), lambda: (0, 0)),
        out_shape=jax.ShapeDtypeStruct((1, 1), jnp.float32),
    )(hist2, labels.reshape(n, 1))
    return out.reshape(())


def kernel(logits, labels):
    conf = _tc_conf(logits)
    conf_flat = conf.reshape(-1)
    lbl_pad = jnp.concatenate(
        [labels, jnp.zeros((_NPAD - _NREAL,), jnp.int32)])

    mesh = plsc.VectorSubcoreMesh(core_axis_name="c", subcore_axis_name="s")
    sck = functools.partial(
        pl.kernel,
        mesh=mesh,
        out_type=jax.ShapeDtypeStruct((2 * _PUB,), jnp.float32),
        compiler_params=pltpu.CompilerParams(needs_layout_passes=False),
        scratch_types=[
            pltpu.VMEM((_CW,), jnp.float32),
            pltpu.VMEM((_CW,), jnp.float32),
            pltpu.VMEM((_CHR,), jnp.int32),
            pltpu.VMEM((_CHR,), jnp.int32),
            pltpu.VMEM((_HTOT,), jnp.float32),
            pltpu.VMEM((_MSL,), jnp.float32),
            pltpu.VMEM((_MSL,), jnp.float32),
            pltpu.VMEM((_PUB,), jnp.float32),
            pltpu.SemaphoreType.DMA,
            pltpu.SemaphoreType.DMA,
            pltpu.VMEM_SHARED((_NS * _PUB,), jnp.float32),
        ],
    )(_sc_kernel)
    hist2 = sck(conf_flat, lbl_pad)
    return _tc_finish(hist2.reshape(2, _PUB), labels)


# speedup vs baseline: 1.3797x; 1.3797x over previous
import functools

import jax
import jax.numpy as jnp
from jax import lax
from jax.experimental import pallas as pl
from jax.experimental.pallas import tpu as pltpu
from jax.experimental.pallas import tpu_sc as plsc

N_BINS = 15
_C = 100
_CP = 112
_NREAL = 50000
_NPAD = 51200
_BN_TC = 1600

_NW = 32
_NS = 16
_RPW = _NPAD // _NW
_CHR = 160
_NCH = _RPW // _CHR
_CW = _CHR * _CP

_SLOT = 16
_HCLS = _CP * _SLOT
_CFS_BASE = _HCLS
_CORR_BASE = 2 * _HCLS
_HTOT = 2 * _HCLS + 16 * _HCLS
_PUB = 3 * _HCLS
_MSL = _PUB // _NS


def _softmax_kernel(logits_ref, conf_ref):
    i = pl.program_id(0)
    x = logits_ref[...]
    bn, c = x.shape
    m = jnp.max(x, axis=1, keepdims=True)
    e = jnp.exp(x - m)
    s = jnp.sum(e, axis=1, keepdims=True)
    conf = e * (1.0 / s)
    row = jax.lax.broadcasted_iota(jnp.int32, (bn, c), 0) + i * bn
    conf = jnp.where(row < _NREAL, conf, 0.0)
    conf_ref[...] = jax.lax.pad(
        conf, jnp.float32(0.0), ((0, 0, 0), (0, _CP - c, 0)))


def _tc_conf(logits):
    n, c = logits.shape
    grid = _NPAD // _BN_TC
    return pl.pallas_call(
        _softmax_kernel,
        grid=(grid,),
        in_specs=[pl.BlockSpec((_BN_TC, c), lambda i: (i, 0))],
        out_specs=pl.BlockSpec((_BN_TC, _CP), lambda i: (i, 0)),
        out_shape=jax.ShapeDtypeStruct((_NPAD, _CP), jnp.float32),
        compiler_params=pltpu.CompilerParams(
            dimension_semantics=("arbitrary",)),
    )(logits)


def _bin_idx(v15):
    ti = v15.astype(jnp.int32)
    tf = ti.astype(jnp.float32)
    one = jnp.ones((16,), jnp.int32)
    return ti - jnp.where(tf == v15, one, jnp.zeros((16,), jnp.int32))


def _sc_kernel(conf_hbm, lbl_hbm, out_hbm,
               cb0, cb1, lb0, lb1, hist, tmp, macc, pub,
               sem0, sem1, shared):
    core = lax.axis_index("c")
    sub = lax.axis_index("s")
    wid = sub * 2 + core
    base_row = wid * _RPW

    iota = lax.iota(jnp.int32, 16)
    ones_f = jnp.ones((16,), jnp.float32)
    zeros_f = jnp.zeros((16,), jnp.float32)

    def _m16(i):
        return pl.multiple_of(i * 16, 16)

    def _z(i, _):
        hist[pl.ds(_m16(i), 16)] = zeros_f
        return 0
    lax.fori_loop(0, _HTOT // 16, _z, 0)

    cbufs = (cb0, cb1)
    lbufs = (lb0, lb1)
    sems = (sem0, sem1)

    def _start(k):
        h = pltpu.async_copy(
            conf_hbm.at[pl.ds(
                pl.multiple_of((base_row + k * _CHR) * _CP, 8), _CW)],
            cbufs[k % 2], sems[k % 2])
        pltpu.sync_copy(
            lbl_hbm.at[pl.ds(pl.multiple_of(base_row + k * _CHR, 8), _CHR)],
            lbufs[k % 2])
        return h

    addr_base = tuple(iota * 16 + (p * 256 + 1) for p in range(7))
    lane_off = iota * _HCLS + (_CORR_BASE + 1)

    handles = {0: _start(0)}
    for k in range(_NCH):
        handles.pop(k).wait()
        if k + 1 < _NCH:
            handles[k + 1] = _start(k + 1)
        cb = cbufs[k % 2]
        lb = lbufs[k % 2]

        @plsc.parallel_loop(0, _CHR, 1, unroll=4)
        def _row(j):
            base = pl.multiple_of(j * _CP, 8)
            for p in range(7):
                v = cb[pl.ds(base + p * 16, 16)]
                bi = _bin_idx(v * jnp.float32(N_BINS))
                addr = addr_base[p] + bi
                plsc.addupdate_scatter(hist, [addr], ones_f)
                plsc.addupdate_scatter(hist, [addr + _CFS_BASE], v)

        @plsc.parallel_loop(0, _CHR // 16, 1, unroll=2)
        def _grp(g):
            lvec = lb[pl.ds(pl.multiple_of(g * 16, 16), 16)]
            rloc = iota + jnp.full((16,), g * 16, jnp.int32)
            word = rloc * _CP + lvec
            cl = plsc.load_gather(cb, [word])
            bi = _bin_idx(cl * jnp.float32(N_BINS))
            caddr = lane_off + lvec * _SLOT + bi
            plsc.addupdate_scatter(hist, [caddr], ones_f)

    def _cmp(i, _):
        o = _m16(i)
        pub[pl.ds(o, 16)] = hist[pl.ds(o, 16)]
        pub[pl.ds(_CFS_BASE + o, 16)] = hist[pl.ds(_CFS_BASE + o, 16)]
        acc = zeros_f
        for lane in range(16):
            acc = acc + hist[pl.ds(_CORR_BASE + lane * _HCLS + o, 16)]
        pub[pl.ds(2 * _HCLS + o, 16)] = acc
        return 0
    lax.fori_loop(0, _HCLS // 16, _cmp, 0)

    pltpu.sync_copy(pub, shared.at[pl.ds(pl.multiple_of(sub * _PUB, 8), _PUB)])
    plsc.subcore_barrier()

    sl = pl.multiple_of(sub * _MSL, 8)

    def _zm(i, _):
        macc[pl.ds(_m16(i), 16)] = zeros_f
        return 0
    lax.fori_loop(0, _MSL // 16, _zm, 0)

    def _mrg(w, _):
        off = pl.multiple_of(w * _PUB + sl, 8)
        pltpu.sync_copy(shared.at[pl.ds(off, _MSL)], tmp)

        def _add(i, _):
            macc[pl.ds(_m16(i), 16)] += tmp[pl.ds(_m16(i), 16)]
            return 0
        lax.fori_loop(0, _MSL // 16, _add, 0)
        return 0
    lax.fori_loop(0, _NS, _mrg, 0)
    plsc.subcore_barrier()

    pltpu.sync_copy(
        macc, out_hbm.at[pl.ds(pl.multiple_of(core * _PUB + sl, 8), _MSL)])


def _finish_kernel(hist_ref, labels_ref, out_ref, *, n_total):
    h = hist_ref[...]
    hs = h[0:1, :] + h[1:2, :]
    cnt = hs[:, 0:_HCLS]
    cfs = hs[:, _HCLS:2 * _HCLS]
    corr = hs[:, 2 * _HCLS:3 * _HCLS]
    lbl = labels_ref[...]
    maxl = jnp.max(lbl)
    num_classes = (maxl + 1).astype(jnp.float32)
    lane = jax.lax.broadcasted_iota(jnp.int32, (1, _HCLS), 1)
    slot = jax.lax.rem(lane, _SLOT)
    cls = jax.lax.div(lane, _SLOT)
    safe = jnp.maximum(cnt, 1.0)
    term = jnp.abs(cfs / safe - corr / safe) * cnt * jnp.float32(1.0 / n_total)
    ok = ((slot >= 1) & (cls < maxl + 1) & (cnt > 0.0)).astype(jnp.float32)
    out_ref[...] = jnp.sum(term * ok, keepdims=True) / num_classes


def _tc_finish(hist2, labels):
    n = labels.shape[0]
    out = pl.pallas_call(
        functools.partial(_finish_kernel, n_total=n),
        in_specs=[
            pl.BlockSpec((2, _PUB), lambda: (0, 0)),
            pl.BlockSpec((n, 1), lambda: (0, 0)),
        ],
        out_specs=pl.BlockSpec((1, 1), lambda: (0, 0)),
        out_shape=jax.ShapeDtypeStruct((1, 1), jnp.float32),
    )(hist2, labels.reshape(n, 1))
    return out.reshape(())


def kernel(logits, labels):
    conf = _tc_conf(logits)
    conf_flat = conf.reshape(-1)
    lbl_pad = jnp.concatenate(
        [labels, jnp.zeros((_NPAD - _NREAL,), jnp.int32)])

    mesh = plsc.VectorSubcoreMesh(core_axis_name="c", subcore_axis_name="s")
    sck = functools.partial(
        pl.kernel,
        mesh=mesh,
        out_type=jax.ShapeDtypeStruct((2 * _PUB,), jnp.float32),
        compiler_params=pltpu.CompilerParams(needs_layout_passes=False),
        scratch_types=[
            pltpu.VMEM((_CW,), jnp.float32),
            pltpu.VMEM((_CW,), jnp.float32),
            pltpu.VMEM((_CHR,), jnp.int32),
            pltpu.VMEM((_CHR,), jnp.int32),
            pltpu.VMEM((_HTOT,), jnp.float32),
            pltpu.VMEM((_MSL,), jnp.float32),
            pltpu.VMEM((_MSL,), jnp.float32),
            pltpu.VMEM((_PUB,), jnp.float32),
            pltpu.SemaphoreType.DMA,
            pltpu.SemaphoreType.DMA,
            pltpu.VMEM_SHARED((_NS * _PUB,), jnp.float32),
        ],
    )(_sc_kernel)
    hist2 = sck(conf_flat, lbl_pad)
    return _tc_finish(hist2.reshape(2, _PUB), labels)


# speedup vs baseline: 1.4155x; 1.0259x over previous
import functools

import jax
import jax.numpy as jnp
from jax import lax
from jax.experimental import pallas as pl
from jax.experimental.pallas import tpu as pltpu
from jax.experimental.pallas import tpu_sc as plsc

N_BINS = 15
_C = 100
_CP = 112
_NREAL = 50000
_NPAD = 51200
_BN_TC = 1600

_NW = 32
_NS = 16
_RPW = _NPAD // _NW
_CHR = 160
_NCH = _RPW // _CHR
_CW = _CHR * _CP

_SLOT = 16
_HCLS = _CP * _SLOT
_CFS_BASE = _HCLS
_CORR_BASE = 2 * _HCLS
_HTOT = 2 * _HCLS + 16 * _HCLS
_PUB = 3 * _HCLS
_MSL = _PUB // _NS


def _softmax_kernel(logits_ref, conf_ref):
    i = pl.program_id(0)
    x = logits_ref[...]
    bn, c = x.shape
    m = jnp.max(x, axis=1, keepdims=True)
    e = jnp.exp(x - m)
    s = jnp.sum(e, axis=1, keepdims=True)
    conf = e * (1.0 / s)
    row = jax.lax.broadcasted_iota(jnp.int32, (bn, c), 0) + i * bn
    conf = jnp.where(row < _NREAL, conf, 0.0)
    conf_ref[...] = jax.lax.pad(
        conf, jnp.float32(0.0), ((0, 0, 0), (0, _CP - c, 0)))


def _tc_conf(logits):
    n, c = logits.shape
    grid = _NPAD // _BN_TC
    return pl.pallas_call(
        _softmax_kernel,
        grid=(grid,),
        in_specs=[pl.BlockSpec((_BN_TC, c), lambda i: (i, 0))],
        out_specs=pl.BlockSpec((_BN_TC, _CP), lambda i: (i, 0)),
        out_shape=jax.ShapeDtypeStruct((_NPAD, _CP), jnp.float32),
        compiler_params=pltpu.CompilerParams(
            dimension_semantics=("arbitrary",)),
    )(logits)


def _bin_idx(v15):
    ti = v15.astype(jnp.int32)
    tf = ti.astype(jnp.float32)
    one = jnp.ones((16,), jnp.int32)
    return ti - jnp.where(tf == v15, one, jnp.zeros((16,), jnp.int32))


def _sc_kernel(conf_hbm, lbl_hbm, out_hbm,
               cb0, cb1, lb0, lb1, hist, tmp, macc, pub,
               sem0, sem1, shared):
    core = lax.axis_index("c")
    sub = lax.axis_index("s")
    wid = sub * 2 + core
    base_row = wid * _RPW

    iota = lax.iota(jnp.int32, 16)
    ones_f = jnp.ones((16,), jnp.float32)
    zeros_f = jnp.zeros((16,), jnp.float32)

    def _m16(i):
        return pl.multiple_of(i * 16, 16)

    def _z(i, _):
        hist[pl.ds(_m16(i), 16)] = zeros_f
        return 0
    lax.fori_loop(0, _HTOT // 16, _z, 0)

    cbufs = (cb0, cb1)
    lbufs = (lb0, lb1)
    sems = (sem0, sem1)

    def _start(k):
        h = pltpu.async_copy(
            conf_hbm.at[pl.ds(
                pl.multiple_of((base_row + k * _CHR) * _CP, 8), _CW)],
            cbufs[k % 2], sems[k % 2])
        pltpu.sync_copy(
            lbl_hbm.at[pl.ds(pl.multiple_of(base_row + k * _CHR, 8), _CHR)],
            lbufs[k % 2])
        return h

    addr_base = tuple(iota * 16 + (p * 256 + 1) for p in range(7))
    lane_off = iota * _HCLS + (_CORR_BASE + 1)

    handles = {0: _start(0)}
    for k in range(_NCH):
        handles.pop(k).wait()
        if k + 1 < _NCH:
            handles[k + 1] = _start(k + 1)
        cb = cbufs[k % 2]
        lb = lbufs[k % 2]

        @plsc.parallel_loop(0, _CHR, 1, unroll=8)
        def _row(j):
            base = pl.multiple_of(j * _CP, 8)
            for p in range(7):
                v = cb[pl.ds(base + p * 16, 16)]
                bi = _bin_idx(v * jnp.float32(N_BINS))
                addr = addr_base[p] + bi
                plsc.addupdate_scatter(hist, [addr], ones_f)
                plsc.addupdate_scatter(hist, [addr + _CFS_BASE], v)

        @plsc.parallel_loop(0, _CHR // 16, 1, unroll=2)
        def _grp(g):
            lvec = lb[pl.ds(pl.multiple_of(g * 16, 16), 16)]
            rloc = iota + jnp.full((16,), g * 16, jnp.int32)
            word = rloc * _CP + lvec
            cl = plsc.load_gather(cb, [word])
            bi = _bin_idx(cl * jnp.float32(N_BINS))
            caddr = lane_off + lvec * _SLOT + bi
            plsc.addupdate_scatter(hist, [caddr], ones_f)

    def _cmp(i, _):
        o = _m16(i)
        pub[pl.ds(o, 16)] = hist[pl.ds(o, 16)]
        pub[pl.ds(_CFS_BASE + o, 16)] = hist[pl.ds(_CFS_BASE + o, 16)]
        acc = zeros_f
        for lane in range(16):
            acc = acc + hist[pl.ds(_CORR_BASE + lane * _HCLS + o, 16)]
        pub[pl.ds(2 * _HCLS + o, 16)] = acc
        return 0
    lax.fori_loop(0, _HCLS // 16, _cmp, 0)

    pltpu.sync_copy(pub, shared.at[pl.ds(pl.multiple_of(sub * _PUB, 8), _PUB)])
    plsc.subcore_barrier()

    sl = pl.multiple_of(sub * _MSL, 8)

    def _zm(i, _):
        macc[pl.ds(_m16(i), 16)] = zeros_f
        return 0
    lax.fori_loop(0, _MSL // 16, _zm, 0)

    def _mrg(w, _):
        off = pl.multiple_of(w * _PUB + sl, 8)
        pltpu.sync_copy(shared.at[pl.ds(off, _MSL)], tmp)

        def _add(i, _):
            macc[pl.ds(_m16(i), 16)] += tmp[pl.ds(_m16(i), 16)]
            return 0
        lax.fori_loop(0, _MSL // 16, _add, 0)
        return 0
    lax.fori_loop(0, _NS, _mrg, 0)
    plsc.subcore_barrier()

    pltpu.sync_copy(
        macc, out_hbm.at[pl.ds(pl.multiple_of(core * _PUB + sl, 8), _MSL)])


def _finish_kernel(hist_ref, labels_ref, out_ref, *, n_total):
    h = hist_ref[...]
    hs = h[0:1, :] + h[1:2, :]
    cnt = hs[:, 0:_HCLS]
    cfs = hs[:, _HCLS:2 * _HCLS]
    corr = hs[:, 2 * _HCLS:3 * _HCLS]
    lbl = labels_ref[...]
    maxl = jnp.max(lbl)
    num_classes = (maxl + 1).astype(jnp.float32)
    lane = jax.lax.broadcasted_iota(jnp.int32, (1, _HCLS), 1)
    slot = jax.lax.rem(lane, _SLOT)
    cls = jax.lax.div(lane, _SLOT)
    safe = jnp.maximum(cnt, 1.0)
    term = jnp.abs(cfs / safe - corr / safe) * cnt * jnp.float32(1.0 / n_total)
    ok = ((slot >= 1) & (cls < maxl + 1) & (cnt > 0.0)).astype(jnp.float32)
    out_ref[...] = jnp.sum(term * ok, keepdims=True) / num_classes


def _tc_finish(hist2, labels):
    n = labels.shape[0]
    out = pl.pallas_call(
        functools.partial(_finish_kernel, n_total=n),
        in_specs=[
            pl.BlockSpec((2, _PUB), lambda: (0, 0)),
            pl.BlockSpec((400, n // 400), lambda: (0, 0)),
        ],
        out_specs=pl.BlockSpec((1, 1), lambda: (0, 0)),
        out_shape=jax.ShapeDtypeStruct((1, 1), jnp.float32),
    )(hist2, labels.reshape(400, n // 400))
    return out.reshape(())


def kernel(logits, labels):
    conf = _tc_conf(logits)
    conf_flat = conf.reshape(-1)
    lbl_pad = jnp.concatenate(
        [labels, jnp.zeros((_NPAD - _NREAL,), jnp.int32)])

    mesh = plsc.VectorSubcoreMesh(core_axis_name="c", subcore_axis_name="s")
    sck = functools.partial(
        pl.kernel,
        mesh=mesh,
        out_type=jax.ShapeDtypeStruct((2 * _PUB,), jnp.float32),
        compiler_params=pltpu.CompilerParams(needs_layout_passes=False),
        scratch_types=[
            pltpu.VMEM((_CW,), jnp.float32),
            pltpu.VMEM((_CW,), jnp.float32),
            pltpu.VMEM((_CHR,), jnp.int32),
            pltpu.VMEM((_CHR,), jnp.int32),
            pltpu.VMEM((_HTOT,), jnp.float32),
            pltpu.VMEM((_MSL,), jnp.float32),
            pltpu.VMEM((_MSL,), jnp.float32),
            pltpu.VMEM((_PUB,), jnp.float32),
            pltpu.SemaphoreType.DMA,
            pltpu.SemaphoreType.DMA,
            pltpu.VMEM_SHARED((_NS * _PUB,), jnp.float32),
        ],
    )(_sc_kernel)
    hist2 = sck(conf_flat, lbl_pad)
    return _tc_finish(hist2.reshape(2, _PUB), labels)


# speedup vs baseline: 2.5270x; 1.7852x over previous
import functools

import jax
import jax.numpy as jnp
from jax.experimental import pallas as pl
from jax.experimental.pallas import tpu as pltpu

N_BINS = 15
_BIN_PAD = 16
_LANES = 128
_WIDE = N_BINS * _LANES


def _ece_kernel(logits_ref, labels_ref, out_ref,
                cnt_ref, cfs_ref, corr_ref, maxlab_ref,
                *, n_total):
    step = pl.program_id(0)
    nsteps = pl.num_programs(0)

    @pl.when(step == 0)
    def _init():
        cnt_ref[...] = jnp.zeros((8, _WIDE), jnp.float32)
        cfs_ref[...] = jnp.zeros((8, _WIDE), jnp.float32)
        corr_ref[...] = jnp.zeros((_BIN_PAD, _LANES), jnp.float32)
        maxlab_ref[0] = 0

    x = logits_ref[...]
    bn, c = x.shape
    m = jnp.max(x, axis=1, keepdims=True)
    e = jnp.exp(x - m)
    s = jnp.sum(e, axis=1, keepdims=True)
    conf = e * (1.0 / s)

    idx = jnp.ceil(conf * jnp.float32(N_BINS)) - 1.0
    idx = jnp.where(conf > 0.0, idx, -1.0)

    pad_cfg = ((0, 0, 0), (0, _LANES - c, 0))
    idx_p = jax.lax.pad(idx, jnp.float32(-1.0), pad_cfg)
    conf_p = jax.lax.pad(conf, jnp.float32(0.0), pad_cfg)

    lbl = labels_ref[...]
    maxlab_ref[0] = jnp.maximum(maxlab_ref[0], jnp.max(lbl))
    cls_iota = jax.lax.broadcasted_iota(jnp.int32, (bn, c), 1)
    onehot = (lbl == cls_iota).astype(jnp.float32)

    conf_label = jnp.sum(conf * onehot, axis=1, keepdims=True)
    idx_lab = jnp.ceil(conf_label * jnp.float32(N_BINS)) - 1.0
    idx_lab = jnp.where(conf_label > 0.0, idx_lab, -1.0)
    bin_iota = jax.lax.broadcasted_iota(jnp.int32, (bn, _BIN_PAD), 1)
    a = (bin_iota == idx_lab.astype(jnp.int32)).astype(jnp.bfloat16)
    corr_part = jax.lax.dot_general(
        a, onehot.astype(jnp.bfloat16),
        dimension_numbers=(((0,), (0,)), ((), ())),
        preferred_element_type=jnp.float32)
    corr_ref[:, 0:c] += corr_part

    cnt_chunks = []
    cfs_chunks = []
    one_bf = jnp.bfloat16(1.0)
    zero_bf = jnp.bfloat16(0.0)
    idx_bf = idx_p.astype(jnp.bfloat16)
    for b in range(N_BINS):
        eq_bf = idx_bf == jnp.bfloat16(b)
        cnt_chunks.append(jnp.where(eq_bf, one_bf, zero_bf))
        eq = idx_p == jnp.float32(b)
        cfs_chunks.append(jnp.where(eq, conf_p, 0.0))
    cnt_wide = jnp.concatenate(cnt_chunks, axis=1)
    cfs_wide = jnp.concatenate(cfs_chunks, axis=1)
    ones = jnp.ones((1, bn), jnp.float32)
    dn = (((1,), (0,)), ((), ()))
    cnt_row = jax.lax.dot_general(ones.astype(jnp.bfloat16), cnt_wide,
                                  dimension_numbers=dn,
                                  preferred_element_type=jnp.float32)
    cfs_row = jax.lax.dot_general(ones, cfs_wide, dimension_numbers=dn,
                                  preferred_element_type=jnp.float32)
    cnt_ref[0:1, :] += cnt_row
    cfs_ref[0:1, :] += cfs_row

    @pl.when(step == nsteps - 1)
    def _finalize():
        count = cnt_ref[0:1, :].reshape(N_BINS, _LANES)
        confsum = cfs_ref[0:1, :].reshape(N_BINS, _LANES)
        corr = corr_ref[0:N_BINS, :]
        num_classes = (maxlab_ref[0] + 1).astype(jnp.float32)
        prop = count * jnp.float32(1.0 / n_total)
        safe = jnp.maximum(count, 1.0)
        acc_in_bin = corr / safe
        avg_conf = confsum / safe
        term = jnp.where(count > 0.0,
                         jnp.abs(avg_conf - acc_in_bin) * prop, 0.0)
        class_sce = jnp.sum(term, axis=0, keepdims=True)
        cls = jax.lax.broadcasted_iota(jnp.int32, (1, _LANES), 1)
        mask = (cls < (maxlab_ref[0] + 1)).astype(jnp.float32)
        out_ref[...] = jnp.sum(class_sce * mask, keepdims=True) / num_classes


def kernel(logits, labels):
    n, c = logits.shape
    bn = n
    for cand in (2000, 1250, 1000, 625, 500, 400, 250, 200, 125, 100):
        if n % cand == 0 and cand % 8 == 0:
            bn = cand
            break
    grid = n // bn
    out = pl.pallas_call(
        functools.partial(_ece_kernel, n_total=n),
        grid=(grid,),
        in_specs=[
            pl.BlockSpec((bn, c), lambda i: (i, 0)),
            pl.BlockSpec((bn, 1), lambda i: (i, 0)),
        ],
        out_specs=pl.BlockSpec((1, 1), lambda i: (0, 0)),
        out_shape=jax.ShapeDtypeStruct((1, 1), jnp.float32),
        scratch_shapes=[
            pltpu.VMEM((8, _WIDE), jnp.float32),
            pltpu.VMEM((8, _WIDE), jnp.float32),
            pltpu.VMEM((_BIN_PAD, _LANES), jnp.float32),
            pltpu.SMEM((1,), jnp.int32),
        ],
        compiler_params=pltpu.CompilerParams(
            dimension_semantics=("arbitrary",)),
    )(logits, labels.reshape(n, 1))
    return out.reshape(())
